# Initial kernel scaffold; baseline (speedup 1.0000x reference)
import jax
import jax.numpy as jnp
from jax.experimental import pallas as pl


def kernel(x, edge_index, eps1, W1, b1, eps2, W2, b2, eps3, W3, b3, Wl1, bl1, Wl2, bl2):
    raise NotImplementedError("write your pallas kernel here")


# speedup vs baseline: 3.5886x; 3.5886x over previous
import functools

import jax
import jax.numpy as jnp
from jax import lax
from jax.experimental import pallas as pl
from jax.experimental.pallas import tpu as pltpu
from jax.experimental.pallas import tpu_sc as plsc

N, E, D, H, C = 10000, 320000, 128, 128, 40
NC, NS, L = 2, 16, 16
NW = NC * NS
K = 128
CPW = -(-(E // K) // NW)
E_PAD = CPW * NW * K
N_ACC = N + L
RT = 624
TAIL0 = NS * RT

_mesh = plsc.VectorSubcoreMesh(core_axis_name="c", subcore_axis_name="s",
                               num_cores=NC, num_subcores=NS)


def _sc_agg_body(y_hbm, src_hbm, dst_hbm, zeros_hbm, out_hbm,
                 acc, sidx, didx, rows, zbuf, sem):
    cid = lax.axis_index("c")
    sid = lax.axis_index("s")
    w = cid * NS + sid

    pltpu.sync_copy(zeros_hbm, zbuf)
    z0 = sid * RT
    nfull = RT // 128
    for i in range(nfull):
        pltpu.sync_copy(zbuf, acc.at[pl.ds(z0 + i * 128, 128)])
    rem = RT - nfull * 128
    if rem:
        pltpu.sync_copy(zbuf.at[pl.ds(0, rem)], acc.at[pl.ds(z0 + nfull * 128, rem)])

    @pl.when(sid == NS - 1)
    def _zero_tail():
        pltpu.sync_copy(zbuf.at[pl.ds(0, N_ACC - TAIL0)], acc.at[pl.ds(TAIL0, N_ACC - TAIL0)])

    plsc.subcore_barrier()

    def chunk(j, carry):
        base = (w * CPW + j) * K
        pltpu.sync_copy(src_hbm.at[pl.ds(base, K)], sidx)
        pltpu.sync_copy(dst_hbm.at[pl.ds(base, K)], didx)
        pltpu.async_copy(y_hbm.at[sidx], rows, sem).wait()
        pltpu.sync_copy(rows, acc.at[didx], add=True)
        return carry

    lax.fori_loop(0, CPW, chunk, 0)
    plsc.subcore_barrier()

    o0 = sid * RT
    ofull = RT // 128
    for i in range(ofull):
        pltpu.sync_copy(acc.at[pl.ds(o0 + i * 128, 128)], rows)
        pltpu.sync_copy(rows, out_hbm.at[pl.ds(cid * N + o0 + i * 128, 128)])
    orem = RT - ofull * 128
    if orem:
        pltpu.sync_copy(acc.at[pl.ds(o0 + ofull * 128, orem)], rows.at[pl.ds(0, orem)])
        pltpu.sync_copy(rows.at[pl.ds(0, orem)],
                        out_hbm.at[pl.ds(cid * N + o0 + ofull * 128, orem)])

    @pl.when(sid == NS - 1)
    def _copy_tail():
        pltpu.sync_copy(acc.at[pl.ds(TAIL0, N - TAIL0)], rows.at[pl.ds(0, N - TAIL0)])
        pltpu.sync_copy(rows.at[pl.ds(0, N - TAIL0)],
                        out_hbm.at[pl.ds(cid * N + TAIL0, N - TAIL0)])


_sc_agg = pl.kernel(
    _sc_agg_body,
    out_type=jax.ShapeDtypeStruct((NC * N, H), jnp.float32),
    mesh=_mesh,
    scratch_types=[
        pltpu.VMEM_SHARED((N_ACC, H), jnp.float32),
        pltpu.VMEM((K,), jnp.int32),
        pltpu.VMEM((K,), jnp.int32),
        pltpu.VMEM((K, H), jnp.float32),
        pltpu.VMEM((128, H), jnp.float32),
        pltpu.SemaphoreType.DMA,
    ],
)

BN = 1000


def _mm_first_body(x_ref, w_ref, o_ref):
    o_ref[...] = jnp.dot(x_ref[...], w_ref[...], preferred_element_type=jnp.float32)


_mm_first = pl.pallas_call(
    _mm_first_body,
    grid=(N // BN,),
    in_specs=[pl.BlockSpec((BN, D), lambda i: (i, 0)),
              pl.BlockSpec((D, H), lambda i: (0, 0))],
    out_specs=pl.BlockSpec((BN, H), lambda i: (i, 0)),
    out_shape=jax.ShapeDtypeStruct((N, H), jnp.float32),
)


def _mid_body(e_ref, y_ref, a0_ref, a1_ref, b_ref, w_ref, o_ref):
    h = e_ref[0] * y_ref[...] + a0_ref[...] + a1_ref[...] + b_ref[...]
    h = jnp.maximum(h, 0.0)
    o_ref[...] = jnp.dot(h, w_ref[...], preferred_element_type=jnp.float32)


_mm_mid = pl.pallas_call(
    _mid_body,
    grid=(N // BN,),
    in_specs=[pl.BlockSpec(memory_space=pltpu.SMEM),
              pl.BlockSpec((BN, H), lambda i: (i, 0)),
              pl.BlockSpec((BN, H), lambda i: (i, 0)),
              pl.BlockSpec((BN, H), lambda i: (i, 0)),
              pl.BlockSpec((1, H), lambda i: (0, 0)),
              pl.BlockSpec((H, H), lambda i: (0, 0))],
    out_specs=pl.BlockSpec((BN, H), lambda i: (i, 0)),
    out_shape=jax.ShapeDtypeStruct((N, H), jnp.float32),
)


def _final_body(e_ref, y_ref, a0_ref, a1_ref, b_ref, wl1_ref, bl1_ref,
                wl2_ref, o_ref):
    h = e_ref[0] * y_ref[...] + a0_ref[...] + a1_ref[...] + b_ref[...]
    h = jnp.maximum(h, 0.0)
    h = jnp.dot(h, wl1_ref[...], preferred_element_type=jnp.float32) + bl1_ref[...]
    h = jnp.maximum(h, 0.0)
    z = jnp.dot(h, wl2_ref[...], preferred_element_type=jnp.float32)
    col = lax.broadcasted_iota(jnp.int32, z.shape, 1)
    zm = jnp.where(col < C, z, -jnp.inf)
    m = jnp.max(zm, axis=1, keepdims=True)
    lse = jnp.log(jnp.sum(jnp.exp(zm - m), axis=1, keepdims=True)) + m
    o_ref[...] = z - lse


_mm_final = pl.pallas_call(
    _final_body,
    grid=(N // BN,),
    in_specs=[pl.BlockSpec(memory_space=pltpu.SMEM),
              pl.BlockSpec((BN, H), lambda i: (i, 0)),
              pl.BlockSpec((BN, H), lambda i: (i, 0)),
              pl.BlockSpec((BN, H), lambda i: (i, 0)),
              pl.BlockSpec((1, H), lambda i: (0, 0)),
              pl.BlockSpec((H, H), lambda i: (0, 0)),
              pl.BlockSpec((1, H), lambda i: (0, 0)),
              pl.BlockSpec((H, 128), lambda i: (0, 0))],
    out_specs=pl.BlockSpec((BN, 128), lambda i: (i, 0)),
    out_shape=jax.ShapeDtypeStruct((N, 128), jnp.float32),
)


def kernel(x, edge_index, eps1, W1, b1, eps2, W2, b2, eps3, W3, b3,
           Wl1, bl1, Wl2, bl2):
    src = edge_index[0]
    dst = edge_index[1]
    pad = E_PAD - E
    src_p = jnp.concatenate([src, jnp.zeros((pad,), jnp.int32)])
    dst_p = jnp.concatenate([dst, jnp.full((pad,), N, jnp.int32)])
    zeros128 = jnp.zeros((128, H), jnp.float32)
    Wl2p = jnp.zeros((H, 128), jnp.float32).at[:, :C].set(Wl2)

    y = _mm_first(x, W1)
    a = _sc_agg(y, src_p, dst_p, zeros128)
    y = _mm_mid(jnp.reshape(1.0 + eps1, (1,)), y, a[:N], a[N:], b1[None, :], W2)
    a = _sc_agg(y, src_p, dst_p, zeros128)
    y = _mm_mid(jnp.reshape(1.0 + eps2, (1,)), y, a[:N], a[N:], b2[None, :], W3)
    a = _sc_agg(y, src_p, dst_p, zeros128)
    out = _mm_final(jnp.reshape(1.0 + eps3, (1,)), y, a[:N], a[N:],
                    b3[None, :], Wl1, bl1[None, :], Wl2p)
    return out[:, :C]


# speedup vs baseline: 5.2231x; 1.4555x over previous
import functools

import jax
import jax.numpy as jnp
from jax import lax
from jax.experimental import pallas as pl
from jax.experimental.pallas import tpu as pltpu
from jax.experimental.pallas import tpu_sc as plsc

N, E, D, H, C = 10000, 320000, 128, 128, 40
NC, NS, L = 2, 16, 16
NW = NC * NS
K = 112
NB = 3
CPW = NB * (-(-(E // K) // (NW * NB)))
NG = CPW // NB
E_PAD = CPW * NW * K
N_ACC = N + L
RT = 624
TAIL0 = NS * RT

_mesh = plsc.VectorSubcoreMesh(core_axis_name="c", subcore_axis_name="s",
                               num_cores=NC, num_subcores=NS)


def _sc_agg_body(y_hbm, src_hbm, dst_hbm, zeros_hbm, out_hbm,
                 acc, sidx, didx, rows, isem, gsem, ssem):
    cid = lax.axis_index("c")
    sid = lax.axis_index("s")
    w = cid * NS + sid

    zbuf = rows.at[0]
    pltpu.sync_copy(zeros_hbm, zbuf)
    z0 = sid * RT
    nfull = RT // K
    for i in range(nfull):
        pltpu.sync_copy(zbuf, acc.at[pl.ds(z0 + i * K, K)])
    rem = RT - nfull * K
    if rem:
        pltpu.sync_copy(zbuf.at[pl.ds(0, rem)], acc.at[pl.ds(z0 + nfull * K, rem)])

    @pl.when(sid == NS - 1)
    def _zero_tail():
        pltpu.sync_copy(zbuf.at[pl.ds(0, N_ACC - TAIL0)], acc.at[pl.ds(TAIL0, N_ACC - TAIL0)])

    plsc.subcore_barrier()

    def idx_load(g, p):
        for b in range(NB):
            base = (w * CPW + g * NB + b) * K
            pltpu.async_copy(src_hbm.at[pl.ds(base, K)], sidx.at[p, b],
                             isem.at[p, 2 * b])
            pltpu.async_copy(dst_hbm.at[pl.ds(base, K)], didx.at[p, b],
                             isem.at[p, 2 * b + 1])

    idx_load(0, 0)

    def group(g, carry):
        p = lax.rem(g, 2)
        for b in range(NB):
            pltpu.make_async_copy(src_hbm.at[pl.ds(0, K)], sidx.at[p, b],
                                  isem.at[p, 2 * b]).wait()
            pltpu.make_async_copy(dst_hbm.at[pl.ds(0, K)], didx.at[p, b],
                                  isem.at[p, 2 * b + 1]).wait()

        @pl.when(g > 0)
        def _wait_prev_scatters():
            for b in range(NB):
                pltpu.make_async_copy(rows.at[b], acc.at[pl.ds(0, K)],
                                      ssem.at[b]).wait()

        for b in range(NB):
            pltpu.async_copy(y_hbm.at[sidx.at[p, b]], rows.at[b], gsem.at[b])

        @pl.when(g + 1 < NG)
        def _prefetch_next_idx():
            idx_load(g + 1, 1 - p)

        for b in range(NB):
            pltpu.make_async_copy(y_hbm.at[sidx.at[p, b]], rows.at[b],
                                  gsem.at[b]).wait()
            pltpu.async_copy(rows.at[b], acc.at[didx.at[p, b]], ssem.at[b],
                             add=True)
        return carry

    lax.fori_loop(0, NG, group, 0)
    for b in range(NB):
        pltpu.make_async_copy(rows.at[b], acc.at[pl.ds(0, K)], ssem.at[b]).wait()
    plsc.subcore_barrier()

    stage = rows.at[0]
    o0 = sid * RT
    ofull = RT // K
    for i in range(ofull):
        pltpu.sync_copy(acc.at[pl.ds(o0 + i * K, K)], stage)
        pltpu.sync_copy(stage, out_hbm.at[pl.ds(cid * N + o0 + i * K, K)])
    orem = RT - ofull * K
    if orem:
        pltpu.sync_copy(acc.at[pl.ds(o0 + ofull * K, orem)], stage.at[pl.ds(0, orem)])
        pltpu.sync_copy(stage.at[pl.ds(0, orem)],
                        out_hbm.at[pl.ds(cid * N + o0 + ofull * K, orem)])

    @pl.when(sid == NS - 1)
    def _copy_tail():
        pltpu.sync_copy(acc.at[pl.ds(TAIL0, N - TAIL0)], stage.at[pl.ds(0, N - TAIL0)])
        pltpu.sync_copy(stage.at[pl.ds(0, N - TAIL0)],
                        out_hbm.at[pl.ds(cid * N + TAIL0, N - TAIL0)])


_sc_agg = pl.kernel(
    _sc_agg_body,
    out_type=jax.ShapeDtypeStruct((NC * N, H), jnp.float32),
    mesh=_mesh,
    scratch_types=[
        pltpu.VMEM_SHARED((N_ACC, H), jnp.float32),
        pltpu.VMEM((2, NB, K), jnp.int32),
        pltpu.VMEM((2, NB, K), jnp.int32),
        pltpu.VMEM((NB, K, H), jnp.float32),
        pltpu.SemaphoreType.DMA((2, 2 * NB)),
        pltpu.SemaphoreType.DMA((NB,)),
        pltpu.SemaphoreType.DMA((NB,)),
    ],
)

BN = 1000


def _mm_first_body(x_ref, w_ref, o_ref):
    o_ref[...] = jnp.dot(x_ref[...], w_ref[...], preferred_element_type=jnp.float32)


_mm_first = pl.pallas_call(
    _mm_first_body,
    grid=(N // BN,),
    in_specs=[pl.BlockSpec((BN, D), lambda i: (i, 0)),
              pl.BlockSpec((D, H), lambda i: (0, 0))],
    out_specs=pl.BlockSpec((BN, H), lambda i: (i, 0)),
    out_shape=jax.ShapeDtypeStruct((N, H), jnp.float32),
)


def _mid_body(e_ref, y_ref, a0_ref, a1_ref, b_ref, w_ref, o_ref):
    h = e_ref[0] * y_ref[...] + a0_ref[...] + a1_ref[...] + b_ref[...]
    h = jnp.maximum(h, 0.0)
    o_ref[...] = jnp.dot(h, w_ref[...], preferred_element_type=jnp.float32)


_mm_mid = pl.pallas_call(
    _mid_body,
    grid=(N // BN,),
    in_specs=[pl.BlockSpec(memory_space=pltpu.SMEM),
              pl.BlockSpec((BN, H), lambda i: (i, 0)),
              pl.BlockSpec((BN, H), lambda i: (i, 0)),
              pl.BlockSpec((BN, H), lambda i: (i, 0)),
              pl.BlockSpec((1, H), lambda i: (0, 0)),
              pl.BlockSpec((H, H), lambda i: (0, 0))],
    out_specs=pl.BlockSpec((BN, H), lambda i: (i, 0)),
    out_shape=jax.ShapeDtypeStruct((N, H), jnp.float32),
)


def _final_body(e_ref, y_ref, a0_ref, a1_ref, b_ref, wl1_ref, bl1_ref,
                wl2_ref, o_ref):
    h = e_ref[0] * y_ref[...] + a0_ref[...] + a1_ref[...] + b_ref[...]
    h = jnp.maximum(h, 0.0)
    h = jnp.dot(h, wl1_ref[...], preferred_element_type=jnp.float32) + bl1_ref[...]
    h = jnp.maximum(h, 0.0)
    z = jnp.dot(h, wl2_ref[...], preferred_element_type=jnp.float32)
    col = lax.broadcasted_iota(jnp.int32, z.shape, 1)
    zm = jnp.where(col < C, z, -jnp.inf)
    m = jnp.max(zm, axis=1, keepdims=True)
    lse = jnp.log(jnp.sum(jnp.exp(zm - m), axis=1, keepdims=True)) + m
    o_ref[...] = z - lse


_mm_final = pl.pallas_call(
    _final_body,
    grid=(N // BN,),
    in_specs=[pl.BlockSpec(memory_space=pltpu.SMEM),
              pl.BlockSpec((BN, H), lambda i: (i, 0)),
              pl.BlockSpec((BN, H), lambda i: (i, 0)),
              pl.BlockSpec((BN, H), lambda i: (i, 0)),
              pl.BlockSpec((1, H), lambda i: (0, 0)),
              pl.BlockSpec((H, H), lambda i: (0, 0)),
              pl.BlockSpec((1, H), lambda i: (0, 0)),
              pl.BlockSpec((H, 128), lambda i: (0, 0))],
    out_specs=pl.BlockSpec((BN, 128), lambda i: (i, 0)),
    out_shape=jax.ShapeDtypeStruct((N, 128), jnp.float32),
)


def kernel(x, edge_index, eps1, W1, b1, eps2, W2, b2, eps3, W3, b3,
           Wl1, bl1, Wl2, bl2):
    src = edge_index[0]
    dst = edge_index[1]
    pad = E_PAD - E
    src_p = jnp.concatenate([src, jnp.zeros((pad,), jnp.int32)])
    dst_p = jnp.concatenate([dst, jnp.full((pad,), N, jnp.int32)])
    zeros128 = jnp.zeros((K, H), jnp.float32)
    Wl2p = jnp.zeros((H, 128), jnp.float32).at[:, :C].set(Wl2)

    y = _mm_first(x, W1)
    a = _sc_agg(y, src_p, dst_p, zeros128)
    y = _mm_mid(jnp.reshape(1.0 + eps1, (1,)), y, a[:N], a[N:], b1[None, :], W2)
    a = _sc_agg(y, src_p, dst_p, zeros128)
    y = _mm_mid(jnp.reshape(1.0 + eps2, (1,)), y, a[:N], a[N:], b2[None, :], W3)
    a = _sc_agg(y, src_p, dst_p, zeros128)
    out = _mm_final(jnp.reshape(1.0 + eps3, (1,)), y, a[:N], a[N:],
                    b3[None, :], Wl1, bl1[None, :], Wl2p)
    return out[:, :C]
